# baseline (speedup 1.0000x reference)
import functools

import jax
import jax.numpy as jnp
from jax import lax
from jax.experimental import pallas as pl
from jax.experimental.pallas import tpu as pltpu

_ROT = ((13, 15, 26, 6), (17, 29, 16, 24))
_K0 = 0
_K1 = 42
_K2 = _K0 ^ _K1 ^ 0x1BD11BDA
_SCHED = ((0, 1, 2, 1), (1, 2, 0, 2), (0, 0, 1, 3), (1, 1, 2, 4), (0, 2, 0, 5))

_LANES = 8192


def _gumbel_bits(flat):
    ks = (jnp.uint32(_K0), jnp.uint32(_K1), jnp.uint32(_K2))
    x0 = jnp.zeros_like(flat) + ks[0]
    x1 = flat + ks[1]
    for rset, ka, kb, inc in _SCHED:
        for r in _ROT[rset]:
            x0 = x0 + x1
            x1 = (x1 << jnp.uint32(r)) | lax.shift_right_logical(
                x1, jnp.uint32(32 - r))
            x1 = x0 ^ x1
        x0 = x0 + ks[ka]
        x1 = x1 + ks[kb] + jnp.uint32(inc)
    return x0 ^ x1


def _sampler_kernel(temps_ref, logits_ref, out_ref,
                    gval, gidx, sval, sidx, *, V, C, NV):
    j = pl.program_id(0)

    @pl.when(j == 0)
    def _init():
        gval[:] = jnp.full_like(gval[:], -jnp.inf)
        gidx[:] = jnp.zeros_like(gidx[:])
        sval[:] = jnp.full_like(sval[:], -jnp.inf)
        sidx[:] = jnp.zeros_like(sidx[:])

    x = logits_ref[:, :]
    B = x.shape[0]
    base = j * C
    lane = lax.broadcasted_iota(jnp.int32, (B, C), 1)
    vglob = lane + base
    valid = vglob < V

    neg_inf = jnp.float32(-jnp.inf)
    big = jnp.int32(2**31 - 1)

    xm = jnp.where(valid, x, neg_inf)
    bm = jnp.max(xm, axis=1, keepdims=True)
    cand = jnp.where(jnp.logical_and(xm == bm, valid), vglob, big)
    bix = jnp.min(cand, axis=1, keepdims=True)
    better = bm > gval[:]
    gval[:] = jnp.where(better, bm, gval[:])
    gidx[:] = jnp.where(better, bix, gidx[:])

    row = lax.broadcasted_iota(jnp.uint32, (B, C), 0)
    flat = row * jnp.uint32(V) + vglob.astype(jnp.uint32)
    bits = _gumbel_bits(flat)
    fb = lax.shift_right_logical(bits, jnp.uint32(9)) | jnp.uint32(0x3F800000)
    f = lax.bitcast_convert_type(fb, jnp.float32) - jnp.float32(1.0)
    tiny = jnp.float32(jnp.finfo(jnp.float32).tiny)
    u = jnp.maximum(tiny, f + tiny)
    g = -jnp.log(-jnp.log(u))

    t = temps_ref[:, :]
    safe_t = jnp.where(t == 0.0, jnp.float32(1.0), t)
    val = jnp.where(valid, g + x / safe_t, neg_inf)
    bm2 = jnp.max(val, axis=1, keepdims=True)
    cand2 = jnp.where(jnp.logical_and(val == bm2, valid), vglob, big)
    bix2 = jnp.min(cand2, axis=1, keepdims=True)
    better2 = bm2 > sval[:]
    sval[:] = jnp.where(better2, bm2, sval[:])
    sidx[:] = jnp.where(better2, bix2, sidx[:])

    @pl.when(j == NV - 1)
    def _fin():
        out_ref[:] = jnp.where(t == 0.0, gidx[:], sidx[:])


def kernel(logits, temperatures):
    B, V = logits.shape
    C = _LANES
    NV = pl.cdiv(V, C)
    temps = temperatures.reshape(B, 1)
    out = pl.pallas_call(
        functools.partial(_sampler_kernel, V=V, C=C, NV=NV),
        grid=(NV,),
        in_specs=[
            pl.BlockSpec((B, 1), lambda j: (0, 0)),
            pl.BlockSpec((B, C), lambda j: (0, j)),
        ],
        out_specs=pl.BlockSpec((B, 1), lambda j: (0, 0)),
        out_shape=jax.ShapeDtypeStruct((B, 1), jnp.int32),
        scratch_shapes=[
            pltpu.VMEM((B, 1), jnp.float32),
            pltpu.VMEM((B, 1), jnp.int32),
            pltpu.VMEM((B, 1), jnp.float32),
            pltpu.VMEM((B, 1), jnp.int32),
        ],
    )(temps, logits)
    return out.reshape(B)


# speedup vs baseline: 1.1795x; 1.1795x over previous
import functools

import jax
import jax.numpy as jnp
from jax import lax
from jax.experimental import pallas as pl
from jax.experimental.pallas import tpu as pltpu

_ROT = ((13, 15, 26, 6), (17, 29, 16, 24))
_K0 = 0
_K1 = 42
_K2 = _K0 ^ _K1 ^ 0x1BD11BDA
_SCHED = ((0, 1, 2, 1), (1, 2, 0, 2), (0, 0, 1, 3), (1, 1, 2, 4), (0, 2, 0, 5))

_LANES = 8192
_SUB = 512


def _gumbel_bits(flat):
    ks = (jnp.uint32(_K0), jnp.uint32(_K1), jnp.uint32(_K2))
    x0 = jnp.zeros_like(flat) + ks[0]
    x1 = flat + ks[1]
    for rset, ka, kb, inc in _SCHED:
        for r in _ROT[rset]:
            x0 = x0 + x1
            x1 = (x1 << jnp.uint32(r)) | lax.shift_right_logical(
                x1, jnp.uint32(32 - r))
            x1 = x0 ^ x1
        x0 = x0 + ks[ka]
        x1 = x1 + ks[kb] + jnp.uint32(inc)
    return x0 ^ x1


def _sampler_kernel(temps_ref, rowv_ref, logits_ref, out_ref,
                    mval, midx, *, V, C, S, NV):
    j = pl.program_id(0)

    @pl.when(j == 0)
    def _init():
        mval[:] = jnp.full_like(mval[:], -jnp.inf)
        midx[:] = jnp.zeros_like(midx[:])

    B = mval.shape[0]
    t = temps_ref[:, :]
    safe_t = jnp.where(t == 0.0, jnp.float32(1.0), t)
    gmask = jnp.where(t == 0.0, jnp.float32(0.0), jnp.float32(1.0))
    rowv = rowv_ref[:, :]
    base = j * C
    neg_inf = jnp.float32(-jnp.inf)
    tiny = jnp.float32(jnp.finfo(jnp.float32).tiny)

    def body(k, carry):
        off = k * S
        x = logits_ref[:, pl.ds(off, S)]
        lane = lax.broadcasted_iota(jnp.int32, (B, S), 1) + (base + off)
        flat = rowv + lane.astype(jnp.uint32)
        bits = _gumbel_bits(flat)
        fb = (lax.shift_right_logical(bits, jnp.uint32(9))
              | jnp.uint32(0x3F800000))
        f = lax.bitcast_convert_type(fb, jnp.float32) - jnp.float32(1.0)
        u = jnp.maximum(tiny, f + tiny)
        g = -jnp.log(-jnp.log(u))
        val = jnp.where(lane < V, g * gmask + x / safe_t, neg_inf)
        m = mval[:, pl.ds(off, S)]
        better = val > m
        mval[:, pl.ds(off, S)] = jnp.where(better, val, m)
        midx[:, pl.ds(off, S)] = jnp.where(
            better, lane, midx[:, pl.ds(off, S)])
        return carry

    lax.fori_loop(0, C // S, body, 0, unroll=False)

    @pl.when(j == NV - 1)
    def _fin():
        m = mval[:]
        best = jnp.max(m, axis=1, keepdims=True)
        cand = jnp.where(m == best, midx[:], jnp.int32(2**31 - 1))
        out_ref[:] = jnp.min(cand, axis=1, keepdims=True)


def kernel(logits, temperatures):
    B, V = logits.shape
    C = _LANES
    NV = pl.cdiv(V, C)
    temps = temperatures.reshape(B, 1)
    rowv = (jnp.arange(B, dtype=jnp.uint32) * jnp.uint32(V)).reshape(B, 1)
    out = pl.pallas_call(
        functools.partial(_sampler_kernel, V=V, C=C, S=_SUB, NV=NV),
        grid=(NV,),
        in_specs=[
            pl.BlockSpec((B, 1), lambda j: (0, 0)),
            pl.BlockSpec((B, 1), lambda j: (0, 0)),
            pl.BlockSpec((B, C), lambda j: (0, j)),
        ],
        out_specs=pl.BlockSpec((B, 1), lambda j: (0, 0)),
        out_shape=jax.ShapeDtypeStruct((B, 1), jnp.int32),
        scratch_shapes=[
            pltpu.VMEM((B, C), jnp.float32),
            pltpu.VMEM((B, C), jnp.int32),
        ],
    )(temps, rowv, logits)
    return out.reshape(B)
